# baseline (speedup 1.0000x reference)
import dataclasses
import functools

import jax
import jax.numpy as jnp
from jax import lax
from jax.experimental import pallas as pl
from jax.experimental.pallas import tpu as pltpu
from jax.experimental.pallas import tpu_sc as plsc

_D = 32
_V = 1000000
_PACK = 4
_SROWS = _V // _PACK


_FIRE = 8


def _sc_gather_native(t3, idx):
    B = idx.shape[0]
    info = plsc.get_sparse_core_info()
    nw = info.num_cores * info.num_subcores
    b_per_w = B // nw
    mesh = plsc.VectorSubcoreMesh(core_axis_name="c", subcore_axis_name="s")
    cp = pltpu.CompilerParams()
    if "needs_layout_passes" in pltpu.CompilerParams.__dataclass_fields__:
        cp = dataclasses.replace(cp, needs_layout_passes=False)

    @functools.partial(
        pl.kernel,
        mesh=mesh,
        compiler_params=cp,
        out_type=jax.ShapeDtypeStruct((B, _D), jnp.float32),
        scratch_types=[
            pltpu.VMEM((b_per_w + 16,), jnp.int32),
            pltpu.VMEM((_FIRE, _D // 8, 8, 128), jnp.float32),
            pltpu.VMEM((b_per_w, _D), jnp.float32),
            pltpu.SemaphoreType.DMA,
        ],
    )
    def k(t_hbm, g_hbm, o_hbm, idx_v, tiles_v, rows_v, sem):
        wid = lax.axis_index("s") * info.num_cores + lax.axis_index("c")
        base = wid * b_per_w
        pltpu.sync_copy(g_hbm.at[pl.ds(base, b_per_w)], idx_v.at[pl.ds(0, b_per_w)])
        sub16 = jax.lax.iota(jnp.int32, 16)
        dt0 = sub16 // 8
        s16 = sub16 % 8

        @pl.loop(0, b_per_w, step=_FIRE)
        def _(i):
            ivec = idx_v[pl.ds(i, 16)]
            copies = []
            for j in range(_FIRE):
                c = ivec[j]
                col0 = (c >> 7) * 128
                copies.append(pltpu.async_copy(
                    t_hbm.at[:, :, pl.ds(col0, 128)], tiles_v.at[j], sem))
            for j, cp in enumerate(copies):
                cp.wait()
                c = ivec[j]
                lane = jnp.full((16,), c & 127, jnp.int32)
                lo = plsc.load_gather(tiles_v.at[j], [dt0, s16, lane])
                hi = plsc.load_gather(tiles_v.at[j], [dt0 + 2, s16, lane])
                rows_v.at[i + j, pl.ds(0, 16)][...] = lo
                rows_v.at[i + j, pl.ds(16, 16)][...] = hi
        pltpu.sync_copy(rows_v, o_hbm.at[pl.ds(base, b_per_w)])

    return k(t3, idx)


def _mlp_body(u_ref, i_ref, w1_ref, b1_ref, w2_ref, b2_ref, w3_ref, b3_ref,
              o_ref):
    h1 = jnp.dot(u_ref[...], w1_ref[0:_D, :], preferred_element_type=jnp.float32)
    h1 = h1 + jnp.dot(i_ref[...], w1_ref[_D:2 * _D, :],
                      preferred_element_type=jnp.float32)
    h1 = jnp.maximum(h1 + b1_ref[...], 0.0)
    h2 = jnp.dot(h1, w2_ref[...], preferred_element_type=jnp.float32)
    h2 = jnp.maximum(h2 + b2_ref[...], 0.0)
    o_ref[...] = jnp.dot(h2, w3_ref[...],
                         preferred_element_type=jnp.float32) + b3_ref[...]


def _tc_mlp_narrow(u_emb, i_emb, W1, b1, W2, b2, W3, b3, interpret=False):
    B = u_emb.shape[0]
    blk = 2048
    n1 = W1.shape[1]
    n2 = W2.shape[1]
    return pl.pallas_call(
        _mlp_body,
        grid=(B // blk,),
        in_specs=[
            pl.BlockSpec((blk, _D), lambda i: (i, 0)),
            pl.BlockSpec((blk, _D), lambda i: (i, 0)),
            pl.BlockSpec((2 * _D, n1), lambda i: (0, 0)),
            pl.BlockSpec((1, n1), lambda i: (0, 0)),
            pl.BlockSpec((n1, n2), lambda i: (0, 0)),
            pl.BlockSpec((1, n2), lambda i: (0, 0)),
            pl.BlockSpec((n2, 1), lambda i: (0, 0)),
            pl.BlockSpec((1, 1), lambda i: (0, 0)),
        ],
        out_specs=pl.BlockSpec((blk, 1), lambda i: (i, 0)),
        out_shape=jax.ShapeDtypeStruct((B, 1), jnp.float32),
        interpret=interpret,
    )(u_emb, i_emb, W1, b1.reshape(1, -1), W2, b2.reshape(1, -1), W3,
      b3.reshape(1, -1))


def kernel(user_input, item_input, user_table, item_table,
           W1, b1, W2, b2, W3, b3):
    B = user_input.shape[0]
    cu = user_input.astype(jnp.int32)
    ci = item_input.astype(jnp.int32)
    tu3 = user_table.T.reshape(_D // 8, 8, _V)
    ti3 = item_table.T.reshape(_D // 8, 8, _V)
    h = B // 2
    outs = []
    for lo in (0, h):
        u_emb = _sc_gather_native(tu3, lax.slice(cu, (lo,), (lo + h,)))
        i_emb = _sc_gather_native(ti3, lax.slice(ci, (lo,), (lo + h,)))
        outs.append(_tc_mlp_narrow(u_emb, i_emb, W1, b1, W2, b2, W3, b3))
    return jnp.concatenate(outs, axis=0)


# speedup vs baseline: 1.0147x; 1.0147x over previous
import dataclasses
import functools

import jax
import jax.numpy as jnp
from jax import lax
from jax.experimental import pallas as pl
from jax.experimental.pallas import tpu as pltpu
from jax.experimental.pallas import tpu_sc as plsc

_D = 32
_V = 1000000
_PACK = 4
_SROWS = _V // _PACK


_FIRE = 8


def _sc_gather_native(t3, idx):
    B = idx.shape[0]
    info = plsc.get_sparse_core_info()
    nw = info.num_cores * info.num_subcores
    b_per_w = B // nw
    mesh = plsc.VectorSubcoreMesh(core_axis_name="c", subcore_axis_name="s")
    cp = pltpu.CompilerParams()
    if "needs_layout_passes" in pltpu.CompilerParams.__dataclass_fields__:
        cp = dataclasses.replace(cp, needs_layout_passes=False)

    @functools.partial(
        pl.kernel,
        mesh=mesh,
        compiler_params=cp,
        out_type=jax.ShapeDtypeStruct((B, _D), jnp.float32),
        scratch_types=[
            pltpu.VMEM((b_per_w + 16,), jnp.int32),
            pltpu.VMEM((_FIRE, _D // 8, 8, 128), jnp.float32),
            pltpu.VMEM((b_per_w, _D), jnp.float32),
            pltpu.SemaphoreType.DMA,
        ],
    )
    def k(t_hbm, g_hbm, o_hbm, idx_v, tiles_v, rows_v, sem):
        wid = lax.axis_index("s") * info.num_cores + lax.axis_index("c")
        base = wid * b_per_w
        pltpu.sync_copy(g_hbm.at[pl.ds(base, b_per_w)], idx_v.at[pl.ds(0, b_per_w)])
        sub16 = jax.lax.iota(jnp.int32, 16)
        dt0 = sub16 // 8
        s16 = sub16 % 8

        @pl.loop(0, b_per_w, step=_FIRE)
        def _(i):
            ivec = idx_v[pl.ds(i, 16)]
            copies = []
            for j in range(_FIRE):
                c = ivec[j]
                col0 = (c >> 7) * 128
                copies.append(pltpu.async_copy(
                    t_hbm.at[:, :, pl.ds(col0, 128)], tiles_v.at[j], sem))
            for j, cp in enumerate(copies):
                cp.wait()
                c = ivec[j]
                lane = jnp.full((16,), c & 127, jnp.int32)
                lo = plsc.load_gather(tiles_v.at[j], [dt0, s16, lane])
                hi = plsc.load_gather(tiles_v.at[j], [dt0 + 2, s16, lane])
                rows_v.at[i + j, pl.ds(0, 16)][...] = lo
                rows_v.at[i + j, pl.ds(16, 16)][...] = hi
        pltpu.sync_copy(rows_v, o_hbm.at[pl.ds(base, b_per_w)])

    return k(t3, idx)


def _mlp_body(u_ref, i_ref, w1_ref, b1_ref, w2_ref, b2_ref, w3_ref, b3_ref,
              o_ref):
    h1 = jnp.dot(u_ref[...], w1_ref[0:_D, :], preferred_element_type=jnp.float32)
    h1 = h1 + jnp.dot(i_ref[...], w1_ref[_D:2 * _D, :],
                      preferred_element_type=jnp.float32)
    h1 = jnp.maximum(h1 + b1_ref[...], 0.0)
    h2 = jnp.dot(h1, w2_ref[...], preferred_element_type=jnp.float32)
    h2 = jnp.maximum(h2 + b2_ref[...], 0.0)
    o_ref[...] = jnp.dot(h2, w3_ref[...],
                         preferred_element_type=jnp.float32) + b3_ref[...]


def _tc_mlp_narrow(u_emb, i_emb, W1, b1, W2, b2, W3, b3, interpret=False):
    B = u_emb.shape[0]
    blk = 2048
    n1 = W1.shape[1]
    n2 = W2.shape[1]
    return pl.pallas_call(
        _mlp_body,
        grid=(B // blk,),
        in_specs=[
            pl.BlockSpec((blk, _D), lambda i: (i, 0)),
            pl.BlockSpec((blk, _D), lambda i: (i, 0)),
            pl.BlockSpec((2 * _D, n1), lambda i: (0, 0)),
            pl.BlockSpec((1, n1), lambda i: (0, 0)),
            pl.BlockSpec((n1, n2), lambda i: (0, 0)),
            pl.BlockSpec((1, n2), lambda i: (0, 0)),
            pl.BlockSpec((n2, 1), lambda i: (0, 0)),
            pl.BlockSpec((1, 1), lambda i: (0, 0)),
        ],
        out_specs=pl.BlockSpec((blk, 1), lambda i: (i, 0)),
        out_shape=jax.ShapeDtypeStruct((B, 1), jnp.float32),
        interpret=interpret,
    )(u_emb, i_emb, W1, b1.reshape(1, -1), W2, b2.reshape(1, -1), W3,
      b3.reshape(1, -1))


def kernel(user_input, item_input, user_table, item_table,
           W1, b1, W2, b2, W3, b3):
    cu = user_input.astype(jnp.int32)
    ci = item_input.astype(jnp.int32)
    tu3 = user_table.T.reshape(_D // 8, 8, _V)
    ti3 = item_table.T.reshape(_D // 8, 8, _V)
    u_emb = _sc_gather_native(tu3, cu)
    i_emb = _sc_gather_native(ti3, ci)
    return _tc_mlp_narrow(u_emb, i_emb, W1, b1, W2, b2, W3, b3)


# speedup vs baseline: 1.0246x; 1.0098x over previous
import dataclasses
import functools

import jax
import jax.numpy as jnp
from jax import lax
from jax.experimental import pallas as pl
from jax.experimental.pallas import tpu as pltpu
from jax.experimental.pallas import tpu_sc as plsc

_D = 32
_V = 1000000
_PACK = 4
_SROWS = _V // _PACK


_FIRE = 8


def _sc_gather_native(t3, idx):
    B = idx.shape[0]
    info = plsc.get_sparse_core_info()
    nw = info.num_cores * info.num_subcores
    b_per_w = B // nw
    mesh = plsc.VectorSubcoreMesh(core_axis_name="c", subcore_axis_name="s")
    cp = pltpu.CompilerParams()
    if "needs_layout_passes" in pltpu.CompilerParams.__dataclass_fields__:
        cp = dataclasses.replace(cp, needs_layout_passes=False)

    @functools.partial(
        pl.kernel,
        mesh=mesh,
        compiler_params=cp,
        out_type=jax.ShapeDtypeStruct((B, _D), jnp.float32),
        scratch_types=[
            pltpu.VMEM((b_per_w + 16,), jnp.int32),
            pltpu.VMEM((_FIRE, _D // 8, 8, 128), jnp.float32),
            pltpu.VMEM((b_per_w, _D), jnp.float32),
            pltpu.SemaphoreType.DMA,
        ],
    )
    def k(t_hbm, g_hbm, o_hbm, idx_v, tiles_v, rows_v, sem):
        wid = lax.axis_index("s") * info.num_cores + lax.axis_index("c")
        base = wid * b_per_w
        pltpu.sync_copy(g_hbm.at[pl.ds(base, b_per_w)], idx_v.at[pl.ds(0, b_per_w)])
        sub16 = jax.lax.iota(jnp.int32, 16)
        dt0 = sub16 // 8
        s16 = sub16 % 8

        @pl.loop(0, b_per_w, step=_FIRE)
        def _(i):
            ivec = idx_v[pl.ds(i, 16)]
            copies = []
            for j in range(_FIRE):
                c = ivec[j]
                col0 = (c >> 7) * 128
                copies.append(pltpu.async_copy(
                    t_hbm.at[:, :, pl.ds(col0, 128)], tiles_v.at[j], sem))
            for j, cp in enumerate(copies):
                cp.wait()
                c = ivec[j]
                lane = jnp.full((16,), c & 127, jnp.int32)
                lo = plsc.load_gather(tiles_v.at[j], [dt0, s16, lane])
                hi = plsc.load_gather(tiles_v.at[j], [dt0 + 2, s16, lane])
                rows_v.at[i + j, pl.ds(0, 16)][...] = lo
                rows_v.at[i + j, pl.ds(16, 16)][...] = hi
        pltpu.sync_copy(rows_v, o_hbm.at[pl.ds(base, b_per_w)])

    return k(t3, idx)


def _mlp_body(u_ref, i_ref, w1_ref, b1_ref, w2_ref, b2_ref, w3_ref, b3_ref,
              o_ref):
    h1 = jnp.dot(u_ref[...], w1_ref[0:_D, :], preferred_element_type=jnp.float32)
    h1 = h1 + jnp.dot(i_ref[...], w1_ref[_D:2 * _D, :],
                      preferred_element_type=jnp.float32)
    h1 = jnp.maximum(h1 + b1_ref[...], 0.0)
    h2 = jnp.dot(h1, w2_ref[...], preferred_element_type=jnp.float32)
    h2 = jnp.maximum(h2 + b2_ref[...], 0.0)
    o_ref[...] = (jnp.dot(h2, w3_ref[...], preferred_element_type=jnp.float32)
                  + b3_ref[...])[:, 0]


def _tc_mlp_narrow(u_emb, i_emb, W1, b1, W2, b2, W3, b3, interpret=False):
    B = u_emb.shape[0]
    blk = 2048
    n1 = W1.shape[1]
    n2 = W2.shape[1]
    return pl.pallas_call(
        _mlp_body,
        grid=(B // blk,),
        in_specs=[
            pl.BlockSpec((blk, _D), lambda i: (i, 0)),
            pl.BlockSpec((blk, _D), lambda i: (i, 0)),
            pl.BlockSpec((2 * _D, n1), lambda i: (0, 0)),
            pl.BlockSpec((1, n1), lambda i: (0, 0)),
            pl.BlockSpec((n1, n2), lambda i: (0, 0)),
            pl.BlockSpec((1, n2), lambda i: (0, 0)),
            pl.BlockSpec((n2, 1), lambda i: (0, 0)),
            pl.BlockSpec((1, 1), lambda i: (0, 0)),
        ],
        out_specs=pl.BlockSpec((blk,), lambda i: (i,)),
        out_shape=jax.ShapeDtypeStruct((B,), jnp.float32),
        interpret=interpret,
    )(u_emb, i_emb, W1, b1.reshape(1, -1), W2, b2.reshape(1, -1), W3,
      b3.reshape(1, -1)).reshape(B, 1)


def kernel(user_input, item_input, user_table, item_table,
           W1, b1, W2, b2, W3, b3):
    cu = user_input.astype(jnp.int32)
    ci = item_input.astype(jnp.int32)
    tu3 = user_table.T.reshape(_D // 8, 8, _V)
    ti3 = item_table.T.reshape(_D // 8, 8, _V)
    u_emb = _sc_gather_native(tu3, cu)
    i_emb = _sc_gather_native(ti3, ci)
    return _tc_mlp_narrow(u_emb, i_emb, W1, b1, W2, b2, W3, b3)
